# Initial kernel scaffold; baseline (speedup 1.0000x reference)
import jax
import jax.numpy as jnp
from jax.experimental import pallas as pl


def kernel(X, edge_index, W):
    raise NotImplementedError("write your pallas kernel here")


# speedup vs baseline: 3.6065x; 3.6065x over previous
import functools

import jax
import jax.numpy as jnp
from jax import lax
from jax.experimental import pallas as pl
from jax.experimental.pallas import tpu as pltpu
from jax.experimental.pallas import tpu_sc as plsc

N = 10000
E = 320000
D = 128

N_PAD = 10240
E_PAD = 327680
K = 128
NW = 32
CH = E_PAD // (NW * K)
ROWS_PER_TILE = N_PAD // 16
_MM_BLOCK = 512


def _mm_body(x_ref, w_ref, o_ref):
    o_ref[...] = jnp.dot(x_ref[...], w_ref[...],
                         preferred_element_type=jnp.float32)


def _add_body(a_ref, b_ref, o_ref):
    o_ref[...] = a_ref[...] + b_ref[...]


def _matmul(x, w):
    return pl.pallas_call(
        _mm_body,
        grid=(N_PAD // _MM_BLOCK,),
        in_specs=[
            pl.BlockSpec((_MM_BLOCK, D), lambda i: (i, 0)),
            pl.BlockSpec((D, D), lambda i: (0, 0)),
        ],
        out_specs=pl.BlockSpec((_MM_BLOCK, D), lambda i: (i, 0)),
        out_shape=jax.ShapeDtypeStruct((N_PAD, D), jnp.float32),
    )(x, w)


def _combine(a, b):
    return pl.pallas_call(
        _add_body,
        grid=(N_PAD // _MM_BLOCK,),
        in_specs=[
            pl.BlockSpec((_MM_BLOCK, D), lambda i: (i, 0)),
            pl.BlockSpec((_MM_BLOCK, D), lambda i: (i, 0)),
        ],
        out_specs=pl.BlockSpec((_MM_BLOCK, D), lambda i: (i, 0)),
        out_shape=jax.ShapeDtypeStruct((N_PAD, D), jnp.float32),
    )(a, b)


_sc_mesh = plsc.VectorSubcoreMesh(core_axis_name="c", subcore_axis_name="s")


@functools.partial(
    pl.kernel,
    mesh=_sc_mesh,
    out_type=jax.ShapeDtypeStruct((2, N_PAD, D), jnp.float32),
    scratch_types=[
        pltpu.VMEM((CH, K), jnp.int32),
        pltpu.VMEM((CH, K), jnp.int32),
        pltpu.VMEM((K, D), jnp.float32),
        pltpu.VMEM_SHARED((N_PAD, D), jnp.float32),
        pltpu.SemaphoreType.DMA,
    ],
)
def _sc_aggregate(xp_hbm, src_hbm, dst_hbm, zeros_hbm, out_hbm,
                  src_v, dst_v, rows_v, acc, sem):
    c = lax.axis_index("c")
    s = lax.axis_index("s")
    w = c * 16 + s

    pltpu.sync_copy(zeros_hbm, acc.at[pl.ds(s * ROWS_PER_TILE, ROWS_PER_TILE)])
    pltpu.sync_copy(src_hbm.at[pl.ds(w * CH, CH)], src_v)
    pltpu.sync_copy(dst_hbm.at[pl.ds(w * CH, CH)], dst_v)
    plsc.subcore_barrier()

    def body(j, carry):
        pltpu.async_copy(xp_hbm.at[src_v.at[j]], rows_v, sem).wait()
        pltpu.sync_copy(rows_v, acc.at[dst_v.at[j]], add=True)
        return carry

    lax.fori_loop(0, CH, body, 0)

    plsc.subcore_barrier()
    pltpu.sync_copy(acc.at[pl.ds(s * ROWS_PER_TILE, ROWS_PER_TILE)],
                    out_hbm.at[c, pl.ds(s * ROWS_PER_TILE, ROWS_PER_TILE)])


def kernel(X, edge_index, W):
    xpad = jnp.zeros((N_PAD, D), jnp.float32).at[:N].set(X)
    xp = _matmul(xpad, W)
    pad = jnp.full((E_PAD - E,), N, dtype=jnp.int32)
    src = jnp.concatenate([edge_index[0], pad]).reshape(NW * CH, K)
    dst = jnp.concatenate([edge_index[1], pad]).reshape(NW * CH, K)
    zeros = jnp.zeros((ROWS_PER_TILE, D), jnp.float32)
    partials = _sc_aggregate(xp, src, dst, zeros)
    out = _combine(partials[0], partials[1])
    return out[:N]


# speedup vs baseline: 4.0540x; 1.1241x over previous
import functools

import jax
import jax.numpy as jnp
from jax import lax
from jax.experimental import pallas as pl
from jax.experimental.pallas import tpu as pltpu
from jax.experimental.pallas import tpu_sc as plsc

N = 10000
E = 320000
D = 128

N_PAD = 10240
E_PAD = 327680
K = 128
NW = 32
CH = E_PAD // (NW * K)
SB = 16
NB = CH // SB
ROWS_PER_TILE = N_PAD // 16
_MM_BLOCK = 512


def _mm_body(x_ref, w_ref, o_ref):
    o_ref[...] = jnp.dot(x_ref[...], w_ref[...],
                         preferred_element_type=jnp.float32)


def _add_body(a_ref, b_ref, o_ref):
    o_ref[...] = a_ref[...] + b_ref[...]


def _matmul(x, w):
    return pl.pallas_call(
        _mm_body,
        grid=(N_PAD // _MM_BLOCK,),
        in_specs=[
            pl.BlockSpec((_MM_BLOCK, D), lambda i: (i, 0)),
            pl.BlockSpec((D, D), lambda i: (0, 0)),
        ],
        out_specs=pl.BlockSpec((_MM_BLOCK, D), lambda i: (i, 0)),
        out_shape=jax.ShapeDtypeStruct((N_PAD, D), jnp.float32),
    )(x, w)


def _combine(a, b):
    return pl.pallas_call(
        _add_body,
        grid=(N_PAD // _MM_BLOCK,),
        in_specs=[
            pl.BlockSpec((_MM_BLOCK, D), lambda i: (i, 0)),
            pl.BlockSpec((_MM_BLOCK, D), lambda i: (i, 0)),
        ],
        out_specs=pl.BlockSpec((_MM_BLOCK, D), lambda i: (i, 0)),
        out_shape=jax.ShapeDtypeStruct((N_PAD, D), jnp.float32),
    )(a, b)


_sc_mesh = plsc.VectorSubcoreMesh(core_axis_name="c", subcore_axis_name="s")


@functools.partial(
    pl.kernel,
    mesh=_sc_mesh,
    out_type=jax.ShapeDtypeStruct((2, N_PAD, D), jnp.float32),
    scratch_types=[
        pltpu.VMEM((SB, K), jnp.int32),
        pltpu.VMEM((SB, K), jnp.int32),
        pltpu.VMEM((K, D), jnp.float32),
        pltpu.VMEM((K, D), jnp.float32),
        pltpu.VMEM_SHARED((N_PAD, D), jnp.float32),
        pltpu.SemaphoreType.DMA,
        pltpu.SemaphoreType.DMA,
    ],
)
def _sc_aggregate(xp_hbm, src_hbm, dst_hbm, zeros_hbm, out_hbm,
                  src_v, dst_v, rows0, rows1, acc, sem0, sem1):
    c = lax.axis_index("c")
    s = lax.axis_index("s")
    w = c * 16 + s

    pltpu.sync_copy(zeros_hbm, acc.at[pl.ds(s * ROWS_PER_TILE, ROWS_PER_TILE)])
    plsc.subcore_barrier()

    def block(n, carry):
        base = w * CH + n * SB
        pltpu.sync_copy(src_hbm.at[pl.ds(base, SB)], src_v)
        pltpu.sync_copy(dst_hbm.at[pl.ds(base, SB)], dst_v)

        pltpu.async_copy(xp_hbm.at[src_v.at[0]], rows0, sem0)
        pltpu.async_copy(xp_hbm.at[src_v.at[1]], rows1, sem1)

        def body(i, carry2):
            for b, rows, sem in ((0, rows0, sem0), (1, rows1, sem1)):
                j = 2 * i + b
                pltpu.make_async_copy(xp_hbm.at[pl.ds(0, K)], rows, sem).wait()
                pltpu.sync_copy(rows, acc.at[dst_v.at[j]], add=True)

                @pl.when(j + 2 < SB)
                def _():
                    pltpu.async_copy(xp_hbm.at[src_v.at[j + 2]], rows, sem)
            return carry2

        lax.fori_loop(0, SB // 2, body, 0)
        return carry

    lax.fori_loop(0, NB, block, 0)

    plsc.subcore_barrier()
    pltpu.sync_copy(acc.at[pl.ds(s * ROWS_PER_TILE, ROWS_PER_TILE)],
                    out_hbm.at[c, pl.ds(s * ROWS_PER_TILE, ROWS_PER_TILE)])


def kernel(X, edge_index, W):
    xpad = jnp.zeros((N_PAD, D), jnp.float32).at[:N].set(X)
    xp = _matmul(xpad, W)
    pad = jnp.full((E_PAD - E,), N, dtype=jnp.int32)
    src = jnp.concatenate([edge_index[0], pad]).reshape(NW * CH, K)
    dst = jnp.concatenate([edge_index[1], pad]).reshape(NW * CH, K)
    zeros = jnp.zeros((ROWS_PER_TILE, D), jnp.float32)
    partials = _sc_aggregate(xp, src, dst, zeros)
    out = _combine(partials[0], partials[1])
    return out[:N]
